# Initial kernel scaffold; baseline (speedup 1.0000x reference)
import jax
import jax.numpy as jnp
from jax.experimental import pallas as pl


def kernel(features, labels, gradients, old_features):
    raise NotImplementedError("write your pallas kernel here")


# speedup vs baseline: 5.4225x; 5.4225x over previous
import jax
import jax.numpy as jnp
from jax.experimental import pallas as pl
from jax.experimental.pallas import tpu as pltpu

N = 320000
D = 128
CP = 128
B = 6400
NB = N // B


def _pass1_body(f_ref, old_ref, g_ref, lab_ref, sbase_ref, csum_ref, ccnt_ref):
    i = pl.program_id(0)
    f = f_ref[...]
    old = old_ref[...]
    g = g_ref[...]
    lab = lab_ref[...]

    diff = f - old
    ls = 1.0 / (1.0 + jnp.mean(diff * diff, axis=1, keepdims=True))
    gn = jnp.sqrt(jnp.sum(g * g, axis=1, keepdims=True))
    gs = 1.0 / (1.0 + jnp.exp(-gn))
    sbase_ref[...] = 0.3 * ls + 0.5 * gs

    iota = jax.lax.broadcasted_iota(jnp.int32, (B, CP), 1)
    oh = (lab == iota)
    oh_bf = oh.astype(jnp.bfloat16)
    csum_p = jax.lax.dot_general(
        oh_bf, f.astype(jnp.bfloat16),
        dimension_numbers=(((0,), (0,)), ((), ())),
        preferred_element_type=jnp.float32)
    ones = jnp.ones((B, 1), dtype=jnp.float32)
    ccnt_p = jax.lax.dot_general(
        oh.astype(jnp.float32), ones,
        dimension_numbers=(((0,), (0,)), ((), ())),
        preferred_element_type=jnp.float32)

    @pl.when(i == 0)
    def _():
        csum_ref[...] = jnp.zeros_like(csum_ref)
        ccnt_ref[...] = jnp.zeros_like(ccnt_ref)

    csum_ref[...] += csum_p
    ccnt_ref[...] += ccnt_p


def _pass2_body(f_ref, lab_ref, sbase_ref, csum_ref, ccnt_ref, comb_ref):
    f = f_ref[...]
    lab = lab_ref[...]
    cnt = ccnt_ref[...]
    centroids = csum_ref[...] / jnp.maximum(cnt, 1.0)

    iota = jax.lax.broadcasted_iota(jnp.int32, (B, CP), 1)
    oh = (lab == iota)
    c_rows = jax.lax.dot_general(
        oh.astype(jnp.bfloat16), centroids.astype(jnp.bfloat16),
        dimension_numbers=(((1,), (0,)), ((), ())),
        preferred_element_type=jnp.float32)
    cnt_rows = jax.lax.dot_general(
        oh.astype(jnp.float32), cnt,
        dimension_numbers=(((1,), (0,)), ((), ())),
        preferred_element_type=jnp.float32)

    dd = f - c_rows
    dist = jnp.sqrt(jnp.sum(dd * dd, axis=1, keepdims=True))
    typ = jnp.where(cnt_rows > 1.0, 1.0 / (1.0 + dist), 1.0)
    comb_ref[...] = sbase_ref[...] + 0.2 * typ


def _softmax_body(x_ref, p_ref):
    x = x_ref[...]
    m = jnp.max(x)
    e = jnp.exp(x - m)
    p_ref[...] = e / jnp.sum(e)


def kernel(features, labels, gradients, old_features):
    lab2d = labels.astype(jnp.int32).reshape(N, 1)

    row_spec = pl.BlockSpec((B, D), lambda i: (i, 0))
    lab_spec = pl.BlockSpec((B, 1), lambda i: (i, 0))
    col_spec = pl.BlockSpec((B, 1), lambda i: (i, 0))
    acc_spec = pl.BlockSpec((CP, D), lambda i: (0, 0))
    cnt_spec = pl.BlockSpec((CP, 1), lambda i: (0, 0))

    sbase, csum, ccnt = pl.pallas_call(
        _pass1_body,
        grid=(NB,),
        in_specs=[row_spec, row_spec, row_spec, lab_spec],
        out_specs=[col_spec, acc_spec, cnt_spec],
        out_shape=[
            jax.ShapeDtypeStruct((N, 1), jnp.float32),
            jax.ShapeDtypeStruct((CP, D), jnp.float32),
            jax.ShapeDtypeStruct((CP, 1), jnp.float32),
        ],
    )(features, old_features, gradients, lab2d)

    comb = pl.pallas_call(
        _pass2_body,
        grid=(NB,),
        in_specs=[row_spec, lab_spec, col_spec, acc_spec, cnt_spec],
        out_specs=col_spec,
        out_shape=jax.ShapeDtypeStruct((N, 1), jnp.float32),
    )(features, lab2d, sbase, csum, ccnt)

    comb2d = comb.reshape(N // D, D)
    probs = pl.pallas_call(
        _softmax_body,
        out_shape=jax.ShapeDtypeStruct((N // D, D), jnp.float32),
    )(comb2d)

    return comb.reshape(N), probs.reshape(N)
